# baseline (speedup 1.0000x reference)
import functools

import jax
import jax.numpy as jnp
from jax import lax
from jax.experimental import pallas as pl
from jax.experimental.pallas import tpu as pltpu, tpu_sc as plsc

MAX_POS = 8192
EMB = 128

_info = plsc.get_sparse_core_info()
_NC, _NS = _info.num_cores, _info.num_subcores
_NW = _NC * _NS

_ROWS, _COLS = 4, 8192
_B = _ROWS * _COLS
_PER_W = _B // _NW
_WPR = _COLS // _PER_W
_GB = 128
_WB = 256
_GPW = _WB // _GB
_NP = _PER_W // _WB
_K = 3


def _make_kernel():
    mesh = plsc.VectorSubcoreMesh(core_axis_name="c", subcore_axis_name="s")

    @functools.partial(
        pl.kernel,
        mesh=mesh,
        out_type=jax.ShapeDtypeStruct((_B, EMB), jnp.float32),
        scratch_types=[
            pltpu.VMEM((_PER_W,), jnp.int32),
        ]
        + [pltpu.VMEM((_WB, EMB), jnp.float32) for _ in range(_K)]
        + [pltpu.SemaphoreType.DMA for _ in range(_K * (_GPW + 1))],
    )
    def gather_kernel(idx_hbm, table_hbm, out_hbm, idx_v, *bufs_and_sems):
        bufs = bufs_and_sems[:_K]
        gsems = bufs_and_sems[_K : _K + _K * _GPW]
        wsems = bufs_and_sems[_K + _K * _GPW :]
        wid = lax.axis_index("s") * _NC + lax.axis_index("c")
        pltpu.sync_copy(
            idx_hbm.at[wid // _WPR, pl.ds((wid % _WPR) * _PER_W, _PER_W)], idx_v
        )

        def start_gathers(p):
            b = p % _K
            return [
                pltpu.async_copy(
                    table_hbm.at[idx_v.at[pl.ds((p * _GPW + g) * _GB, _GB)]],
                    bufs[b].at[pl.ds(g * _GB, _GB)],
                    gsems[b * _GPW + g],
                )
                for g in range(_GPW)
            ]

        def start_write(p):
            b = p % _K
            base = wid * _PER_W + p * _WB
            return pltpu.async_copy(bufs[b], out_hbm.at[pl.ds(base, _WB)], wsems[b])

        gathers = {p: start_gathers(p) for p in range(_K)}
        writes = {}
        for p in range(_NP):
            for c in gathers.pop(p):
                c.wait()
            if p - 1 in writes and p - 1 + _K < _NP:
                writes.pop(p - 1).wait()
                gathers[p - 1 + _K] = start_gathers(p - 1 + _K)
            writes[p] = start_write(p)
        for p in sorted(writes):
            writes.pop(p).wait()

    return gather_kernel


_gather = _make_kernel()


def kernel(position_ids, table):
    out = _gather(position_ids.astype(jnp.int32), table)
    return out.reshape(position_ids.shape + (EMB,))
